# baseline (speedup 1.0000x reference)
import functools

import jax
import jax.numpy as jnp
from jax import lax
from jax.experimental import pallas as pl
from jax.experimental.pallas import tpu as pltpu
from jax.experimental.pallas import tpu_sc as plsc

B = 512
D = 128
N_ATOMS = 10000
N_BOND_ROWS = 320000
N_BONDS = N_BOND_ROWS // 2

NC = 2
NS = 16
NW = NC * NS

CHUNK = 128
ATOM_PT = 384
BOND_PT = 10112
NA_CHUNKS = ATOM_PT // CHUNK
NB_CHUNKS = BOND_PT // CHUNK

ATOM_LAST = N_ATOMS - CHUNK
BOND_LAST = N_BOND_ROWS - CHUNK

DUMMY = B
ACC_ROWS = 528

_mesh = plsc.VectorSubcoreMesh(core_axis_name="c", subcore_axis_name="s")


@functools.partial(
    pl.kernel,
    out_type=[
        jax.ShapeDtypeStruct((NC, B, D), jnp.float32),
        jax.ShapeDtypeStruct((NC, B, D), jnp.float32),
    ],
    mesh=_mesh,
    scratch_types=[
        pltpu.VMEM((NA_CHUNKS, CHUNK), jnp.int32),
        pltpu.VMEM((NB_CHUNKS, CHUNK), jnp.int32),
        pltpu.VMEM((CHUNK, D), jnp.float32),
        pltpu.VMEM((CHUNK, D), jnp.float32),
        pltpu.VMEM((33, D), jnp.float32),
        pltpu.VMEM_SHARED((ACC_ROWS, D), jnp.float32),
        pltpu.VMEM_SHARED((ACC_ROWS, D), jnp.float32),
        pltpu.SemaphoreType.DMA,
        pltpu.SemaphoreType.DMA,
    ],
)
def _sc_pool(atom_hbm, bond_hbm, aid_hbm, bid_hbm, pa_hbm, pb_hbm,
             aids_v, bids_v, buf0, buf1, zbuf, acc_a, acc_b, semA, semB):
    cid = lax.axis_index("c")
    sid = lax.axis_index("s")
    wid = cid * NS + sid

    zvec = jnp.zeros((16,), jnp.float32)
    for r in range(33):
        for g in range(D // 16):
            zbuf[r, pl.ds(g * 16, 16)] = zvec
    pltpu.sync_copy(zbuf, acc_a.at[pl.ds(sid * 33, 33)])
    pltpu.sync_copy(zbuf, acc_b.at[pl.ds(sid * 33, 33)])

    pltpu.sync_copy(aid_hbm.at[wid], aids_v)
    pltpu.sync_copy(bid_hbm.at[wid], bids_v)

    def astart(j):
        return pl.multiple_of(jnp.minimum(wid * ATOM_PT + j * CHUNK, ATOM_LAST), 8)

    def bstart(j):
        return pl.multiple_of(jnp.minimum(wid * BOND_PT + j * CHUNK, BOND_LAST), 8)

    plsc.subcore_barrier()

    a0 = pltpu.async_copy(atom_hbm.at[pl.ds(astart(0), CHUNK)], buf0, semA)
    a1 = pltpu.async_copy(atom_hbm.at[pl.ds(astart(1), CHUNK)], buf1, semB)
    a0.wait()
    pltpu.sync_copy(buf0, acc_a.at[aids_v.at[0]], add=True)
    a2 = pltpu.async_copy(atom_hbm.at[pl.ds(astart(2), CHUNK)], buf0, semA)
    a1.wait()
    pltpu.sync_copy(buf1, acc_a.at[aids_v.at[1]], add=True)
    b1 = pltpu.async_copy(bond_hbm.at[pl.ds(bstart(1), CHUNK)], buf1, semB)
    a2.wait()
    pltpu.sync_copy(buf0, acc_a.at[aids_v.at[2]], add=True)
    pltpu.async_copy(bond_hbm.at[pl.ds(bstart(0), CHUNK)], buf0, semA)

    def bond_pair(p, carry):
        j = 2 * p
        pltpu.make_async_copy(
            bond_hbm.at[pl.ds(bstart(j), CHUNK)], buf0, semA).wait()
        pltpu.sync_copy(buf0, acc_b.at[bids_v.at[j]], add=True)
        pltpu.async_copy(bond_hbm.at[pl.ds(bstart(j + 2), CHUNK)], buf0, semA)
        pltpu.make_async_copy(
            bond_hbm.at[pl.ds(bstart(j + 1), CHUNK)], buf1, semB).wait()
        pltpu.sync_copy(buf1, acc_b.at[bids_v.at[j + 1]], add=True)
        pltpu.async_copy(bond_hbm.at[pl.ds(bstart(j + 3), CHUNK)], buf1, semB)
        return carry

    lax.fori_loop(0, (NB_CHUNKS - 3) // 2, bond_pair, 0)
    j = NB_CHUNKS - 3
    pltpu.make_async_copy(
        bond_hbm.at[pl.ds(bstart(j), CHUNK)], buf0, semA).wait()
    pltpu.sync_copy(buf0, acc_b.at[bids_v.at[j]], add=True)
    pltpu.async_copy(bond_hbm.at[pl.ds(bstart(j + 2), CHUNK)], buf0, semA)
    pltpu.make_async_copy(
        bond_hbm.at[pl.ds(bstart(j + 1), CHUNK)], buf1, semB).wait()
    pltpu.sync_copy(buf1, acc_b.at[bids_v.at[j + 1]], add=True)
    pltpu.make_async_copy(
        bond_hbm.at[pl.ds(bstart(j + 2), CHUNK)], buf0, semA).wait()
    pltpu.sync_copy(buf0, acc_b.at[bids_v.at[j + 2]], add=True)

    plsc.subcore_barrier()

    pltpu.sync_copy(acc_a.at[pl.ds(sid * 32, 32)], zbuf.at[pl.ds(0, 32)])
    pltpu.sync_copy(zbuf.at[pl.ds(0, 32)], pa_hbm.at[cid, pl.ds(sid * 32, 32)])
    pltpu.sync_copy(acc_b.at[pl.ds(sid * 32, 32)], zbuf.at[pl.ds(0, 32)])
    pltpu.sync_copy(zbuf.at[pl.ds(0, 32)], pb_hbm.at[cid, pl.ds(sid * 32, 32)])


def _combine_body(pa_ref, pb_ref, g_ref, o_ref):
    o_ref[:, 0:D] = pa_ref[0] + pa_ref[1]
    o_ref[:, D:2 * D] = pb_ref[0] + pb_ref[1]
    o_ref[:, 2 * D:3 * D] = g_ref[:]


_combine = pl.pallas_call(
    _combine_body,
    out_shape=jax.ShapeDtypeStruct((B, 3 * D), jnp.float32),
)


def _laid_out_ids(ids, n_rows, n_chunks, last_start, stride=1):
    c = jnp.arange(NW * n_chunks, dtype=jnp.int32)
    k = jnp.arange(CHUNK, dtype=jnp.int32)
    read_row = jnp.minimum(c * CHUNK, last_start)[:, None] + k[None, :]
    prev_end = jnp.minimum(c * CHUNK, n_rows)[:, None]
    contributes = (read_row >= prev_end) & (read_row % stride == 0)
    dummy = DUMMY + (read_row % 16)
    laid = jnp.where(contributes,
                     ids[jnp.clip(read_row // stride, 0, ids.shape[0] - 1)],
                     dummy)
    return laid.reshape(NW, n_chunks, CHUNK)


def kernel(atom_feats, bond_feats, global_feats, atom_segment_ids, bond_segment_ids):
    aid = atom_segment_ids.astype(jnp.int32)
    bid = bond_segment_ids.astype(jnp.int32)
    aid_p = _laid_out_ids(aid, N_ATOMS, NA_CHUNKS, ATOM_LAST)
    bid_p = _laid_out_ids(bid, N_BOND_ROWS, NB_CHUNKS, BOND_LAST, stride=2)
    pa, pb = _sc_pool(atom_feats, bond_feats, aid_p, bid_p)
    return _combine(pa, pb, global_feats)


# speedup vs baseline: 12.2421x; 12.2421x over previous
import functools

import jax
import jax.numpy as jnp
from jax import lax
from jax.experimental import pallas as pl
from jax.experimental.pallas import tpu as pltpu
from jax.experimental.pallas import tpu_sc as plsc

B = 512
D = 128
N_ATOMS = 10000
N_BOND_ROWS = 320000
N_BONDS = N_BOND_ROWS // 2

NC = 2
NS = 16
NW = NC * NS

CHUNK = 128
HALF = CHUNK // 2
ATOM_PT = 384
BOND_PT = 10112
NA_CHUNKS = ATOM_PT // CHUNK
NB_CHUNKS = BOND_PT // CHUNK

ATOM_LAST = N_ATOMS - CHUNK
BOND_LAST = N_BOND_ROWS - CHUNK

DUMMY = B
ACC_ROWS = 528

_mesh = plsc.VectorSubcoreMesh(core_axis_name="c", subcore_axis_name="s")


@functools.partial(
    pl.kernel,
    out_type=[
        jax.ShapeDtypeStruct((NC, B, D), jnp.float32),
        jax.ShapeDtypeStruct((NC, B, D), jnp.float32),
    ],
    mesh=_mesh,
    scratch_types=[
        pltpu.VMEM((NA_CHUNKS, CHUNK), jnp.int32),
        pltpu.VMEM((NB_CHUNKS + 1, HALF), jnp.int32),
        pltpu.VMEM((CHUNK, D), jnp.float32),
        pltpu.VMEM((CHUNK, D), jnp.float32),
        pltpu.VMEM((HALF, D), jnp.float32),
        pltpu.VMEM((HALF, D), jnp.float32),
        pltpu.VMEM((33, D), jnp.float32),
        pltpu.VMEM_SHARED((ACC_ROWS, D), jnp.float32),
        pltpu.VMEM_SHARED((ACC_ROWS, D), jnp.float32),
        pltpu.SemaphoreType.DMA,
        pltpu.SemaphoreType.DMA,
        pltpu.SemaphoreType.DMA,
        pltpu.SemaphoreType.DMA,
    ],
)
def _sc_pool(atom_hbm, bond_hbm, aid_hbm, bid_hbm, pa_hbm, pb_hbm,
             aids_v, bids_v, buf0, buf1, cbuf0, cbuf1, zbuf, acc_a, acc_b,
             semA, semB, semS0, semS1):
    cid = lax.axis_index("c")
    sid = lax.axis_index("s")
    wid = cid * NS + sid

    zvec = jnp.zeros((16,), jnp.float32)
    for r in range(33):
        for g in range(D // 16):
            zbuf[r, pl.ds(g * 16, 16)] = zvec
    pltpu.sync_copy(zbuf, acc_a.at[pl.ds(sid * 33, 33)])
    pltpu.sync_copy(zbuf, acc_b.at[pl.ds(sid * 33, 33)])

    pltpu.sync_copy(aid_hbm.at[wid], aids_v)
    pltpu.sync_copy(bid_hbm.at[wid], bids_v)

    def astart(j):
        return pl.multiple_of(jnp.minimum(wid * ATOM_PT + j * CHUNK, ATOM_LAST), 8)

    def bstart(j):
        return pl.multiple_of(jnp.minimum(wid * BOND_PT + j * CHUNK, BOND_LAST), 8)

    def extract_evens(src, dst):
        def step(i, carry):
            for r in range(4):
                for g in range(D // 16):
                    dst[4 * i + r, pl.ds(g * 16, 16)] = (
                        src[8 * i + 2 * r, pl.ds(g * 16, 16)])
            return carry
        lax.fori_loop(0, HALF // 4, step, 0)

    plsc.subcore_barrier()

    a0 = pltpu.async_copy(atom_hbm.at[pl.ds(astart(0), CHUNK)], buf0, semA)
    a1 = pltpu.async_copy(atom_hbm.at[pl.ds(astart(1), CHUNK)], buf1, semB)
    a0.wait()
    pltpu.sync_copy(buf0, acc_a.at[aids_v.at[0]], add=True)
    a2 = pltpu.async_copy(atom_hbm.at[pl.ds(astart(2), CHUNK)], buf0, semA)
    a1.wait()
    pltpu.sync_copy(buf1, acc_a.at[aids_v.at[1]], add=True)
    pltpu.async_copy(bond_hbm.at[pl.ds(bstart(1), CHUNK)], buf1, semB)
    a2.wait()
    pltpu.sync_copy(buf0, acc_a.at[aids_v.at[2]], add=True)
    pltpu.async_copy(bond_hbm.at[pl.ds(bstart(0), CHUNK)], buf0, semA)

    pltpu.async_copy(cbuf0, acc_b.at[bids_v.at[NB_CHUNKS]], semS0, add=True)
    pltpu.async_copy(cbuf1, acc_b.at[bids_v.at[NB_CHUNKS]], semS1, add=True)

    def bond_pair(p, carry):
        j = 2 * p
        pltpu.make_async_copy(bond_hbm.at[pl.ds(bstart(j), CHUNK)],
                              buf0, semA).wait()
        pltpu.make_async_copy(cbuf0, acc_b.at[bids_v.at[0]], semS0).wait()
        extract_evens(buf0, cbuf0)
        pltpu.async_copy(bond_hbm.at[pl.ds(bstart(j + 2), CHUNK)], buf0, semA)
        pltpu.async_copy(cbuf0, acc_b.at[bids_v.at[j]], semS0, add=True)
        pltpu.make_async_copy(bond_hbm.at[pl.ds(bstart(j + 1), CHUNK)],
                              buf1, semB).wait()
        pltpu.make_async_copy(cbuf1, acc_b.at[bids_v.at[0]], semS1).wait()
        extract_evens(buf1, cbuf1)
        pltpu.async_copy(bond_hbm.at[pl.ds(bstart(j + 3), CHUNK)], buf1, semB)
        pltpu.async_copy(cbuf1, acc_b.at[bids_v.at[j + 1]], semS1, add=True)
        return carry

    lax.fori_loop(0, (NB_CHUNKS - 3) // 2, bond_pair, 0)
    j = NB_CHUNKS - 3
    pltpu.make_async_copy(bond_hbm.at[pl.ds(bstart(j), CHUNK)], buf0, semA).wait()
    pltpu.make_async_copy(cbuf0, acc_b.at[bids_v.at[0]], semS0).wait()
    extract_evens(buf0, cbuf0)
    pltpu.async_copy(bond_hbm.at[pl.ds(bstart(j + 2), CHUNK)], buf0, semA)
    pltpu.async_copy(cbuf0, acc_b.at[bids_v.at[j]], semS0, add=True)
    pltpu.make_async_copy(bond_hbm.at[pl.ds(bstart(j + 1), CHUNK)], buf1, semB).wait()
    pltpu.make_async_copy(cbuf1, acc_b.at[bids_v.at[0]], semS1).wait()
    extract_evens(buf1, cbuf1)
    pltpu.async_copy(cbuf1, acc_b.at[bids_v.at[j + 1]], semS1, add=True)
    pltpu.make_async_copy(bond_hbm.at[pl.ds(bstart(j + 2), CHUNK)], buf0, semA).wait()
    pltpu.make_async_copy(cbuf0, acc_b.at[bids_v.at[0]], semS0).wait()
    extract_evens(buf0, cbuf0)
    pltpu.async_copy(cbuf0, acc_b.at[bids_v.at[j + 2]], semS0, add=True)
    pltpu.make_async_copy(cbuf0, acc_b.at[bids_v.at[0]], semS0).wait()
    pltpu.make_async_copy(cbuf1, acc_b.at[bids_v.at[0]], semS1).wait()

    plsc.subcore_barrier()

    pltpu.sync_copy(acc_a.at[pl.ds(sid * 32, 32)], zbuf.at[pl.ds(0, 32)])
    pltpu.sync_copy(zbuf.at[pl.ds(0, 32)], pa_hbm.at[cid, pl.ds(sid * 32, 32)])
    pltpu.sync_copy(acc_b.at[pl.ds(sid * 32, 32)], zbuf.at[pl.ds(0, 32)])
    pltpu.sync_copy(zbuf.at[pl.ds(0, 32)], pb_hbm.at[cid, pl.ds(sid * 32, 32)])


def _combine_body(pa_ref, pb_ref, g_ref, o_ref):
    o_ref[:, 0:D] = pa_ref[0] + pa_ref[1]
    o_ref[:, D:2 * D] = pb_ref[0] + pb_ref[1]
    o_ref[:, 2 * D:3 * D] = g_ref[:]


_combine = pl.pallas_call(
    _combine_body,
    out_shape=jax.ShapeDtypeStruct((B, 3 * D), jnp.float32),
)


def _atom_ids_laid(aid):
    c = jnp.arange(NW * NA_CHUNKS, dtype=jnp.int32)
    k = jnp.arange(CHUNK, dtype=jnp.int32)
    read_row = jnp.minimum(c * CHUNK, ATOM_LAST)[:, None] + k[None, :]
    prev_end = jnp.minimum(c * CHUNK, N_ATOMS)[:, None]
    dummy = DUMMY + (read_row % 16)
    laid = jnp.where(read_row >= prev_end,
                     aid[jnp.clip(read_row, 0, N_ATOMS - 1)], dummy)
    return laid.reshape(NW, NA_CHUNKS, CHUNK)


def _bond_ids_laid(bid):
    c = jnp.arange(NW * NB_CHUNKS, dtype=jnp.int32)
    k = jnp.arange(HALF, dtype=jnp.int32)
    bond_idx = (jnp.minimum(c * CHUNK, BOND_LAST) // 2)[:, None] + k[None, :]
    real = (c * CHUNK <= BOND_LAST)[:, None]
    dummy = DUMMY + ((c[:, None] + k[None, :]) % 16)
    laid = jnp.where(real, bid[jnp.clip(bond_idx, 0, N_BONDS - 1)], dummy)
    laid = laid.reshape(NW, NB_CHUNKS, HALF)
    extra = jnp.broadcast_to(DUMMY + (k % 16), (NW, 1, HALF)).astype(jnp.int32)
    return jnp.concatenate([laid, extra], axis=1)


def kernel(atom_feats, bond_feats, global_feats, atom_segment_ids, bond_segment_ids):
    aid = atom_segment_ids.astype(jnp.int32)
    bid = bond_segment_ids.astype(jnp.int32)
    aid_p = _atom_ids_laid(aid)
    bid_p = _bond_ids_laid(bid)
    pa, pb = _sc_pool(atom_feats, bond_feats, aid_p, bid_p)
    return _combine(pa, pb, global_feats)


# speedup vs baseline: 14.6604x; 1.1975x over previous
import functools

import jax
import jax.numpy as jnp
from jax import lax
from jax.experimental import pallas as pl
from jax.experimental.pallas import tpu as pltpu
from jax.experimental.pallas import tpu_sc as plsc

B = 512
D = 128
N_ATOMS = 10000
N_BOND_ROWS = 320000
N_BONDS = N_BOND_ROWS // 2

NC = 2
NS = 16
NW = NC * NS

ACHUNK = 128
BCHUNK = 256
BHALF = BCHUNK // 2
ATOM_PT = 384
BOND_PT = 10240
NA_CHUNKS = ATOM_PT // ACHUNK
NB_CHUNKS = BOND_PT // BCHUNK

ATOM_LAST = N_ATOMS - ACHUNK
BOND_LAST = N_BOND_ROWS - BCHUNK

DUMMY = B
ACC_ROWS = 528

_mesh = plsc.VectorSubcoreMesh(core_axis_name="c", subcore_axis_name="s")


@functools.partial(
    pl.kernel,
    out_type=[
        jax.ShapeDtypeStruct((NC, B, D), jnp.float32),
        jax.ShapeDtypeStruct((NC, B, D), jnp.float32),
    ],
    mesh=_mesh,
    scratch_types=[
        pltpu.VMEM((NA_CHUNKS, ACHUNK), jnp.int32),
        pltpu.VMEM((NB_CHUNKS + 1, BHALF), jnp.int32),
        pltpu.VMEM((BCHUNK, D), jnp.float32),
        pltpu.VMEM((BCHUNK, D), jnp.float32),
        pltpu.VMEM((BHALF, D), jnp.float32),
        pltpu.VMEM((BHALF, D), jnp.float32),
        pltpu.VMEM((33, D), jnp.float32),
        pltpu.VMEM_SHARED((ACC_ROWS, D), jnp.float32),
        pltpu.VMEM_SHARED((ACC_ROWS, D), jnp.float32),
        pltpu.SemaphoreType.DMA,
        pltpu.SemaphoreType.DMA,
        pltpu.SemaphoreType.DMA,
        pltpu.SemaphoreType.DMA,
    ],
)
def _sc_pool(atom_hbm, bond_hbm, aid_hbm, bid_hbm, pa_hbm, pb_hbm,
             aids_v, bids_v, buf0, buf1, cbuf0, cbuf1, zbuf, acc_a, acc_b,
             semA, semB, semS0, semS1):
    cid = lax.axis_index("c")
    sid = lax.axis_index("s")
    wid = cid * NS + sid

    zvec = jnp.zeros((16,), jnp.float32)
    for r in range(33):
        for g in range(D // 16):
            zbuf[r, pl.ds(g * 16, 16)] = zvec
    pltpu.sync_copy(zbuf, acc_a.at[pl.ds(sid * 33, 33)])
    pltpu.sync_copy(zbuf, acc_b.at[pl.ds(sid * 33, 33)])

    pltpu.sync_copy(aid_hbm.at[wid], aids_v)
    pltpu.sync_copy(bid_hbm.at[wid], bids_v)

    def astart(j):
        return pl.multiple_of(jnp.minimum(wid * ATOM_PT + j * ACHUNK, ATOM_LAST), 8)

    def bstart(j):
        return pl.multiple_of(jnp.minimum(wid * BOND_PT + j * BCHUNK, BOND_LAST), 8)

    def extract_evens(src, dst):
        def step(i, carry):
            for r in range(8):
                for g in range(D // 16):
                    dst[8 * i + r, pl.ds(g * 16, 16)] = (
                        src[16 * i + 2 * r, pl.ds(g * 16, 16)])
            return carry
        lax.fori_loop(0, BHALF // 8, step, 0)

    plsc.subcore_barrier()

    a0 = pltpu.async_copy(atom_hbm.at[pl.ds(astart(0), ACHUNK)],
                          buf0.at[pl.ds(0, ACHUNK)], semA)
    a1 = pltpu.async_copy(atom_hbm.at[pl.ds(astart(1), ACHUNK)],
                          buf1.at[pl.ds(0, ACHUNK)], semB)
    a0.wait()
    pltpu.sync_copy(buf0.at[pl.ds(0, ACHUNK)], acc_a.at[aids_v.at[0]], add=True)
    a2 = pltpu.async_copy(atom_hbm.at[pl.ds(astart(2), ACHUNK)],
                          buf0.at[pl.ds(0, ACHUNK)], semA)
    a1.wait()
    pltpu.sync_copy(buf1.at[pl.ds(0, ACHUNK)], acc_a.at[aids_v.at[1]], add=True)
    pltpu.async_copy(bond_hbm.at[pl.ds(bstart(1), BCHUNK)], buf1, semB)
    a2.wait()
    pltpu.sync_copy(buf0.at[pl.ds(0, ACHUNK)], acc_a.at[aids_v.at[2]], add=True)
    pltpu.async_copy(bond_hbm.at[pl.ds(bstart(0), BCHUNK)], buf0, semA)

    pltpu.async_copy(cbuf0, acc_b.at[bids_v.at[NB_CHUNKS]], semS0, add=True)
    pltpu.async_copy(cbuf1, acc_b.at[bids_v.at[NB_CHUNKS]], semS1, add=True)

    def bond_pair(p, carry):
        j = 2 * p
        pltpu.make_async_copy(bond_hbm.at[pl.ds(bstart(j), BCHUNK)],
                              buf0, semA).wait()
        pltpu.make_async_copy(cbuf0, acc_b.at[bids_v.at[0]], semS0).wait()
        extract_evens(buf0, cbuf0)
        pltpu.async_copy(bond_hbm.at[pl.ds(bstart(j + 2), BCHUNK)], buf0, semA)
        pltpu.async_copy(cbuf0, acc_b.at[bids_v.at[j]], semS0, add=True)
        pltpu.make_async_copy(bond_hbm.at[pl.ds(bstart(j + 1), BCHUNK)],
                              buf1, semB).wait()
        pltpu.make_async_copy(cbuf1, acc_b.at[bids_v.at[0]], semS1).wait()
        extract_evens(buf1, cbuf1)
        pltpu.async_copy(bond_hbm.at[pl.ds(bstart(j + 3), BCHUNK)], buf1, semB)
        pltpu.async_copy(cbuf1, acc_b.at[bids_v.at[j + 1]], semS1, add=True)
        return carry

    lax.fori_loop(0, NB_CHUNKS // 2 - 1, bond_pair, 0)
    j = NB_CHUNKS - 2
    pltpu.make_async_copy(bond_hbm.at[pl.ds(bstart(j), BCHUNK)], buf0, semA).wait()
    pltpu.make_async_copy(cbuf0, acc_b.at[bids_v.at[0]], semS0).wait()
    extract_evens(buf0, cbuf0)
    pltpu.async_copy(cbuf0, acc_b.at[bids_v.at[j]], semS0, add=True)
    pltpu.make_async_copy(bond_hbm.at[pl.ds(bstart(j + 1), BCHUNK)], buf1, semB).wait()
    pltpu.make_async_copy(cbuf1, acc_b.at[bids_v.at[0]], semS1).wait()
    extract_evens(buf1, cbuf1)
    pltpu.async_copy(cbuf1, acc_b.at[bids_v.at[j + 1]], semS1, add=True)
    pltpu.make_async_copy(cbuf0, acc_b.at[bids_v.at[0]], semS0).wait()
    pltpu.make_async_copy(cbuf1, acc_b.at[bids_v.at[0]], semS1).wait()

    plsc.subcore_barrier()

    pltpu.sync_copy(acc_a.at[pl.ds(sid * 32, 32)], zbuf.at[pl.ds(0, 32)])
    pltpu.sync_copy(zbuf.at[pl.ds(0, 32)], pa_hbm.at[cid, pl.ds(sid * 32, 32)])
    pltpu.sync_copy(acc_b.at[pl.ds(sid * 32, 32)], zbuf.at[pl.ds(0, 32)])
    pltpu.sync_copy(zbuf.at[pl.ds(0, 32)], pb_hbm.at[cid, pl.ds(sid * 32, 32)])


def _combine_body(pa_ref, pb_ref, g_ref, o_ref):
    o_ref[:, 0:D] = pa_ref[0] + pa_ref[1]
    o_ref[:, D:2 * D] = pb_ref[0] + pb_ref[1]
    o_ref[:, 2 * D:3 * D] = g_ref[:]


_combine = pl.pallas_call(
    _combine_body,
    out_shape=jax.ShapeDtypeStruct((B, 3 * D), jnp.float32),
)


def _dummy_chunks(n_chunks, width):
    v = DUMMY + (jnp.arange(n_chunks, dtype=jnp.int32) % 16)
    return jnp.repeat(v, width)


def _atom_ids_laid(aid):
    boundary = NA_CHUNKS * NW * ACHUNK
    n_tail = (boundary - N_ATOMS - (ACHUNK - 16)) // ACHUNK
    laid = jnp.concatenate([
        aid[:ATOM_LAST + ACHUNK - 16],
        jnp.full((ACHUNK - 16,), DUMMY, jnp.int32),
        aid[ATOM_LAST + ACHUNK - 16:],
        _dummy_chunks(n_tail, ACHUNK),
    ])
    return laid.reshape(NW, NA_CHUNKS, ACHUNK)


def _bond_ids_laid(bid):
    total = NW * NB_CHUNKS * BHALF
    n_tail = (total - N_BONDS) // BHALF
    laid = jnp.concatenate([bid, _dummy_chunks(n_tail, BHALF)])
    laid = laid.reshape(NW, NB_CHUNKS, BHALF)
    extra = DUMMY + (jnp.arange(NW, dtype=jnp.int32) % 16)
    extra = jnp.broadcast_to(extra[:, None, None], (NW, 1, BHALF))
    return jnp.concatenate([laid, extra], axis=1)


def kernel(atom_feats, bond_feats, global_feats, atom_segment_ids, bond_segment_ids):
    aid = atom_segment_ids.astype(jnp.int32)
    bid = bond_segment_ids.astype(jnp.int32)
    aid_p = _atom_ids_laid(aid)
    bid_p = _bond_ids_laid(bid)
    pa, pb = _sc_pool(atom_feats, bond_feats, aid_p, bid_p)
    return _combine(pa, pb, global_feats)


# speedup vs baseline: 22.2990x; 1.5210x over previous
import functools

import jax
import jax.numpy as jnp
from jax import lax
from jax.experimental import pallas as pl
from jax.experimental.pallas import tpu as pltpu
from jax.experimental.pallas import tpu_sc as plsc

B = 512
D = 128
N_ATOMS = 10000
N_BOND_ROWS = 320000
N_BONDS = N_BOND_ROWS // 2

NC = 2
NS = 16
NW = NC * NS

BHALF = 128
BOND_PT = 5120
NB_CHUNKS = BOND_PT // BHALF
BOND_LAST = N_BONDS - BHALF

DUMMY = B
ACC_ROWS = 520

_mesh = plsc.VectorSubcoreMesh(core_axis_name="c", subcore_axis_name="s")


@functools.partial(
    pl.kernel,
    out_type=jax.ShapeDtypeStruct((NC, B, D), jnp.float32),
    mesh=_mesh,
    scratch_types=[
        pltpu.VMEM((NB_CHUNKS, BHALF), jnp.int32),
        pltpu.VMEM((BHALF, 2, D), jnp.float32),
        pltpu.VMEM((BHALF, 2, D), jnp.float32),
        pltpu.VMEM((32, D), jnp.float32),
        pltpu.VMEM((32, 2, D), jnp.float32),
        pltpu.VMEM_SHARED((ACC_ROWS, 2, D), jnp.float32),
        pltpu.SemaphoreType.DMA,
        pltpu.SemaphoreType.DMA,
    ],
)
def _sc_bond_pool(bond_hbm, bid_hbm, pb_hbm,
                  bids_v, buf0, buf1, ebuf, zbufb, acc_b, semA, semB):
    cid = lax.axis_index("c")
    sid = lax.axis_index("s")
    wid = cid * NS + sid

    zvec = jnp.zeros((16,), jnp.float32)
    for r in range(32):
        for h in range(2):
            for g in range(D // 16):
                zbufb[r, h, pl.ds(g * 16, 16)] = zvec
    pltpu.sync_copy(zbufb, acc_b.at[pl.ds(sid * 32, 32)])

    @pl.when(sid == 0)
    def _zero_dummy_rows():
        pltpu.sync_copy(zbufb.at[pl.ds(0, 8)], acc_b.at[pl.ds(B, 8)])

    pltpu.sync_copy(bid_hbm.at[wid], bids_v)

    def bstart(j):
        return pl.multiple_of(jnp.minimum(wid * BOND_PT + j * BHALF, BOND_LAST), 8)

    plsc.subcore_barrier()

    pltpu.async_copy(bond_hbm.at[pl.ds(bstart(0), BHALF)], buf0, semA)
    pltpu.async_copy(bond_hbm.at[pl.ds(bstart(1), BHALF)], buf1, semB)

    def bond_pair(p, carry):
        j = 2 * p
        pltpu.make_async_copy(bond_hbm.at[pl.ds(bstart(j), BHALF)],
                              buf0, semA).wait()
        pltpu.sync_copy(buf0, acc_b.at[bids_v.at[j]], add=True)
        pltpu.async_copy(bond_hbm.at[pl.ds(bstart(j + 2), BHALF)], buf0, semA)
        pltpu.make_async_copy(bond_hbm.at[pl.ds(bstart(j + 1), BHALF)],
                              buf1, semB).wait()
        pltpu.sync_copy(buf1, acc_b.at[bids_v.at[j + 1]], add=True)
        pltpu.async_copy(bond_hbm.at[pl.ds(bstart(j + 3), BHALF)], buf1, semB)
        return carry

    lax.fori_loop(0, NB_CHUNKS // 2 - 1, bond_pair, 0)
    j = NB_CHUNKS - 2
    pltpu.make_async_copy(bond_hbm.at[pl.ds(bstart(j), BHALF)], buf0, semA).wait()
    pltpu.sync_copy(buf0, acc_b.at[bids_v.at[j]], add=True)
    pltpu.make_async_copy(bond_hbm.at[pl.ds(bstart(j + 1), BHALF)], buf1, semB).wait()
    pltpu.sync_copy(buf1, acc_b.at[bids_v.at[j + 1]], add=True)

    plsc.subcore_barrier()

    pltpu.sync_copy(acc_b.at[pl.ds(sid * 32, 32)], zbufb)
    for r in range(32):
        for g in range(D // 16):
            ebuf[r, pl.ds(g * 16, 16)] = zbufb[r, 0, pl.ds(g * 16, 16)]
    pltpu.sync_copy(ebuf, pb_hbm.at[cid, pl.ds(sid * 32, 32)])


def _combine_body(atom_ref, aid_ref, pb_ref, g_ref, o_ref):
    seg = lax.broadcasted_iota(jnp.int32, (B, N_ATOMS), 0)
    one_hot = jnp.where(aid_ref[:] == seg, 1.0, 0.0).astype(jnp.float32)
    o_ref[:, 0:D] = jax.lax.dot(one_hot, atom_ref[:],
                                preferred_element_type=jnp.float32)
    o_ref[:, D:2 * D] = pb_ref[0] + pb_ref[1]
    o_ref[:, 2 * D:3 * D] = g_ref[:]


_combine = pl.pallas_call(
    _combine_body,
    out_shape=jax.ShapeDtypeStruct((B, 3 * D), jnp.float32),
)


def _dummy_chunks(n_chunks, width):
    v = DUMMY + (jnp.arange(n_chunks, dtype=jnp.int32) % 8)
    return jnp.repeat(v, width)


def _bond_ids_laid(bid):
    total = NW * NB_CHUNKS * BHALF
    n_tail = (total - N_BONDS) // BHALF
    laid = jnp.concatenate([bid, _dummy_chunks(n_tail, BHALF)])
    return laid.reshape(NW, NB_CHUNKS, BHALF)


def kernel(atom_feats, bond_feats, global_feats, atom_segment_ids, bond_segment_ids):
    aid = atom_segment_ids.astype(jnp.int32)
    bid = bond_segment_ids.astype(jnp.int32)
    bid_p = _bond_ids_laid(bid)
    bond3 = bond_feats.reshape(N_BONDS, 2, D)
    pb = _sc_bond_pool(bond3, bid_p)
    return _combine(atom_feats, aid.reshape(1, N_ATOMS), pb, global_feats)


# speedup vs baseline: 24.2202x; 1.0862x over previous
import functools

import jax
import jax.numpy as jnp
from jax import lax
from jax.experimental import pallas as pl
from jax.experimental.pallas import tpu as pltpu
from jax.experimental.pallas import tpu_sc as plsc

B = 512
D = 128
N_ATOMS = 10000
N_BOND_ROWS = 320000
N_BONDS = N_BOND_ROWS // 2

NC = 2
NS = 16
NW = NC * NS

BHALF = 128
BOND_PT = 5120
NB_CHUNKS = BOND_PT // BHALF
BOND_LAST = N_BONDS - BHALF

DUMMY = B
ACC_ROWS = 520

_mesh = plsc.VectorSubcoreMesh(core_axis_name="c", subcore_axis_name="s")


@functools.partial(
    pl.kernel,
    out_type=jax.ShapeDtypeStruct((NC, B, D), jnp.float32),
    mesh=_mesh,
    scratch_types=[
        pltpu.VMEM((NB_CHUNKS, BHALF), jnp.int32),
        pltpu.VMEM((BHALF, 2, D), jnp.float32),
        pltpu.VMEM((BHALF, 2, D), jnp.float32),
        pltpu.VMEM((BHALF, 2, D), jnp.float32),
        pltpu.VMEM((32, D), jnp.float32),
        pltpu.VMEM((32, 2, D), jnp.float32),
        pltpu.VMEM_SHARED((ACC_ROWS, 2, D), jnp.float32),
        pltpu.SemaphoreType.DMA,
        pltpu.SemaphoreType.DMA,
        pltpu.SemaphoreType.DMA,
    ],
)
def _sc_bond_pool(bond_hbm, bid_hbm, pb_hbm,
                  bids_v, buf0, buf1, buf2, ebuf, zbufb, acc_b, semA, semB, semC):
    cid = lax.axis_index("c")
    sid = lax.axis_index("s")
    wid = cid * NS + sid

    zvec = jnp.zeros((16,), jnp.float32)
    for r in range(32):
        for h in range(2):
            for g in range(D // 16):
                zbufb[r, h, pl.ds(g * 16, 16)] = zvec
    pltpu.sync_copy(zbufb, acc_b.at[pl.ds(sid * 32, 32)])

    @pl.when(sid == 0)
    def _zero_dummy_rows():
        pltpu.sync_copy(zbufb.at[pl.ds(0, 8)], acc_b.at[pl.ds(B, 8)])

    pltpu.sync_copy(bid_hbm.at[wid], bids_v)

    def bstart(j):
        return pl.multiple_of(jnp.minimum(wid * BOND_PT + j * BHALF, BOND_LAST), 8)

    plsc.subcore_barrier()

    pltpu.async_copy(bond_hbm.at[pl.ds(bstart(0), BHALF)], buf0, semA)
    pltpu.async_copy(bond_hbm.at[pl.ds(bstart(1), BHALF)], buf1, semB)
    pltpu.async_copy(bond_hbm.at[pl.ds(bstart(2), BHALF)], buf2, semC)

    def bond_triple(p, carry):
        j = 3 * p
        for off, (buf, sem) in enumerate(((buf0, semA), (buf1, semB),
                                          (buf2, semC))):
            pltpu.make_async_copy(bond_hbm.at[pl.ds(bstart(j + off), BHALF)],
                                  buf, sem).wait()
            pltpu.sync_copy(buf, acc_b.at[bids_v.at[j + off]], add=True)
            pltpu.async_copy(bond_hbm.at[pl.ds(bstart(j + off + 3), BHALF)],
                             buf, sem)
        return carry

    lax.fori_loop(0, (NB_CHUNKS - 4) // 3, bond_triple, 0)
    j = NB_CHUNKS - 4
    pltpu.make_async_copy(bond_hbm.at[pl.ds(bstart(j), BHALF)], buf0, semA).wait()
    pltpu.sync_copy(buf0, acc_b.at[bids_v.at[j]], add=True)
    pltpu.async_copy(bond_hbm.at[pl.ds(bstart(j + 3), BHALF)], buf0, semA)
    pltpu.make_async_copy(bond_hbm.at[pl.ds(bstart(j + 1), BHALF)], buf1, semB).wait()
    pltpu.sync_copy(buf1, acc_b.at[bids_v.at[j + 1]], add=True)
    pltpu.make_async_copy(bond_hbm.at[pl.ds(bstart(j + 2), BHALF)], buf2, semC).wait()
    pltpu.sync_copy(buf2, acc_b.at[bids_v.at[j + 2]], add=True)
    pltpu.make_async_copy(bond_hbm.at[pl.ds(bstart(j + 3), BHALF)], buf0, semA).wait()
    pltpu.sync_copy(buf0, acc_b.at[bids_v.at[j + 3]], add=True)

    plsc.subcore_barrier()

    pltpu.sync_copy(acc_b.at[pl.ds(sid * 32, 32)], zbufb)
    for r in range(32):
        for g in range(D // 16):
            ebuf[r, pl.ds(g * 16, 16)] = zbufb[r, 0, pl.ds(g * 16, 16)]
    pltpu.sync_copy(ebuf, pb_hbm.at[cid, pl.ds(sid * 32, 32)])


def _combine_body(atom_ref, aid_ref, pb_ref, g_ref, o_ref):
    seg = lax.broadcasted_iota(jnp.int32, (B, N_ATOMS), 0)
    one_hot = jnp.where(aid_ref[:] == seg, 1.0, 0.0).astype(jnp.float32)
    o_ref[:, 0:D] = jax.lax.dot(one_hot, atom_ref[:],
                                preferred_element_type=jnp.float32)
    o_ref[:, D:2 * D] = pb_ref[0] + pb_ref[1]
    o_ref[:, 2 * D:3 * D] = g_ref[:]


_combine = pl.pallas_call(
    _combine_body,
    out_shape=jax.ShapeDtypeStruct((B, 3 * D), jnp.float32),
)


def _dummy_chunks(n_chunks, width):
    v = DUMMY + (jnp.arange(n_chunks, dtype=jnp.int32) % 8)
    return jnp.repeat(v, width)


def _bond_ids_laid(bid):
    total = NW * NB_CHUNKS * BHALF
    n_tail = (total - N_BONDS) // BHALF
    laid = jnp.concatenate([bid, _dummy_chunks(n_tail, BHALF)])
    return laid.reshape(NW, NB_CHUNKS, BHALF)


def kernel(atom_feats, bond_feats, global_feats, atom_segment_ids, bond_segment_ids):
    aid = atom_segment_ids.astype(jnp.int32)
    bid = bond_segment_ids.astype(jnp.int32)
    bid_p = _bond_ids_laid(bid)
    bond3 = bond_feats.reshape(N_BONDS, 2, D)
    pb = _sc_bond_pool(bond3, bid_p)
    return _combine(atom_feats, aid.reshape(1, N_ATOMS), pb, global_feats)
